# baseline (speedup 1.0000x reference)
import functools

import jax
import jax.numpy as jnp
from jax import lax
from jax.experimental import pallas as pl
from jax.experimental.pallas import tpu as pltpu
from jax.experimental.pallas import tpu_sc as plsc

SCALE = 8.0
NW = 32
LANES = 16
NBUF = 4
C = 128


def kernel(input, table):
    R, S = input.shape
    V, D = table.shape
    NT8 = D // 8
    TILE = 8 * C

    mesh = plsc.VectorSubcoreMesh(core_axis_name="c", subcore_axis_name="s")

    @functools.partial(
        pl.kernel,
        mesh=mesh,
        out_type=jax.ShapeDtypeStruct((S, NT8, NW, TILE), jnp.float32),
        scratch_types=[
            pltpu.VMEM((C, S), jnp.int32),
            [pltpu.VMEM((C,), jnp.int32) for _ in range(NBUF)],
            [pltpu.VMEM((C, D), jnp.float32) for _ in range(NBUF)],
            [pltpu.VMEM((NT8 * TILE,), jnp.float32) for _ in range(NBUF)],
            [pltpu.SemaphoreType.DMA for _ in range(NBUF)],
            [pltpu.SemaphoreType.DMA for _ in range(NBUF)],
        ],
        compiler_params=pltpu.CompilerParams(
            use_tc_tiling_on_sc=False, needs_layout_passes=False),
    )
    def emb(idx_hbm, table_hbm, out_hbm, idx_v, sidx, gbufs, obufs,
            gsems, ssems):
        wid = lax.axis_index("s") * 2 + lax.axis_index("c")
        pltpu.sync_copy(idx_hbm.at[pl.ds(wid * C, C)], idx_v)

        lanes = lax.iota(jnp.int32, LANES)
        dbase = [
            (lax.shift_right_logical(lanes + 16 * b, 3) * TILE
             + (lanes + 16 * b & 7) * C)
            for b in range(D // LANES)
        ]

        def fire(c, t):
            def col_body(g, carry):
                rows = lanes + g * LANES
                col = jnp.full((LANES,), 0, jnp.int32) + c
                vals = plsc.load_gather(idx_v, [rows, col])
                sidx[t][pl.ds(g * LANES, LANES)] = vals
                return carry
            lax.fori_loop(0, C // LANES, col_body, 0)
            pltpu.async_copy(table_hbm.at[sidx[t]], gbufs[t], gsems[t])

        def gdrain(t):
            pltpu.make_async_copy(
                table_hbm.at[pl.ds(0, C)], gbufs[t], gsems[t]).wait()

        def sdrain(t):
            pltpu.make_async_copy(
                out_hbm.at[0, 0, 0], obufs[t].at[pl.ds(0, TILE)],
                ssems[t]).wait()

        def process(t):
            def row_body(r, carry):
                for b in range(D // LANES):
                    vals = gbufs[t][r, pl.ds(b * LANES, LANES)] * SCALE
                    plsc.store_scatter(obufs[t], [dbase[b] + r], vals)
                return carry
            lax.fori_loop(0, C, row_body, 0)

        def scatter(c, t):
            for t8 in range(NT8):
                pltpu.async_copy(
                    obufs[t].at[pl.ds(t8 * TILE, TILE)],
                    out_hbm.at[c, t8, wid], ssems[t])

        for t in range(NBUF - 1):
            fire(t, t)

        def body(i, carry):
            for t in range(NBUF):
                c = i * NBUF + t
                gdrain(t)
                process(t)
                scatter(c, t)
                nt = (t + NBUF - 1) % NBUF
                nc = c + NBUF - 1

                @pl.when(jnp.logical_and(c >= 1, nc <= S - 1))
                def _():
                    for _k in range(NT8):
                        sdrain(nt)

                @pl.when(nc <= S - 1)
                def _():
                    fire(nc, nt)
            return carry

        lax.fori_loop(0, S // NBUF, body, 0)
        for t in range(NBUF):
            for _k in range(NT8):
                sdrain(t)

    out = emb(input, table)
    out = out.reshape(S, NT8, NW, 8, C)
    out = out.transpose(2, 4, 0, 1, 3)
    return out.reshape(R, S, D)


# speedup vs baseline: 1.0740x; 1.0740x over previous
import functools

import jax
import jax.numpy as jnp
from jax import lax
from jax.experimental import pallas as pl
from jax.experimental.pallas import tpu as pltpu
from jax.experimental.pallas import tpu_sc as plsc

SCALE = 8.0
NW = 32
LANES = 16
NBUF = 2
C = 128


def kernel(input, table):
    R, S = input.shape
    B = R * S
    V, D = table.shape
    BW = B // NW
    NCHUNK = BW // C

    idx = input.reshape(B // C, C)
    tbl = table.reshape(V // 2, 2 * D)

    mesh = plsc.VectorSubcoreMesh(core_axis_name="c", subcore_axis_name="s")

    @functools.partial(
        pl.kernel,
        mesh=mesh,
        out_type=jax.ShapeDtypeStruct((B, D), jnp.float32),
        scratch_types=[
            pltpu.VMEM((NCHUNK, C), jnp.int32),
            [pltpu.VMEM((C,), jnp.int32) for _ in range(NBUF)],
            [pltpu.VMEM((C, 2 * D), jnp.float32) for _ in range(NBUF)],
            [pltpu.VMEM((C, D), jnp.float32) for _ in range(NBUF)],
            [pltpu.SemaphoreType.DMA for _ in range(NBUF)],
            [pltpu.SemaphoreType.DMA for _ in range(NBUF)],
        ],
        compiler_params=pltpu.CompilerParams(use_tc_tiling_on_sc=True),
    )
    def emb(idx_hbm, tbl_hbm, out_hbm, idx_v, sidx, gbufs, obufs,
            gsems, ssems):
        wid = lax.axis_index("s") * 2 + lax.axis_index("c")
        base = wid * BW
        pltpu.sync_copy(idx_hbm.at[pl.ds(wid * NCHUNK, NCHUNK)], idx_v)

        def fire(c, t):
            for g in range(C // LANES):
                sl = pl.ds(g * LANES, LANES)
                sidx[t][sl] = lax.shift_right_logical(idx_v[c, sl], 1)
            pltpu.async_copy(tbl_hbm.at[sidx[t]], gbufs[t], gsems[t])

        def gdrain(t):
            pltpu.make_async_copy(
                tbl_hbm.at[pl.ds(0, C)], gbufs[t], gsems[t]).wait()

        def sdrain(t):
            pltpu.make_async_copy(
                out_hbm.at[pl.ds(0, C)], obufs[t], ssems[t]).wait()

        def process(c, t):
            def group_body(g, carry):
                offv = (idx_v[c, pl.ds(g * LANES, LANES)] & 1) * D
                for k in range(LANES):
                    r = g * LANES + k
                    off = offv[k]
                    for s in range(D // LANES):
                        src = pl.ds(off + s * LANES, LANES)
                        dst = pl.ds(s * LANES, LANES)
                        obufs[t][r, dst] = gbufs[t][r, src] * SCALE
                return carry
            lax.fori_loop(0, C // LANES, group_body, 0)

        for t in range(NBUF - 1):
            fire(t, t)

        def body(i, carry):
            for t in range(NBUF):
                c = i * NBUF + t
                gdrain(t)
                process(c, t)
                pltpu.async_copy(
                    obufs[t], out_hbm.at[pl.ds(base + c * C, C)], ssems[t])
                nt = (t + NBUF - 1) % NBUF
                nc = c + NBUF - 1

                @pl.when(jnp.logical_and(c >= 1, nc <= NCHUNK - 1))
                def _():
                    sdrain(nt)

                @pl.when(nc <= NCHUNK - 1)
                def _():
                    fire(nc, nt)
            return carry

        lax.fori_loop(0, NCHUNK // NBUF, body, 0)
        for t in range(NBUF):
            sdrain(t)

    out = emb(idx, tbl)
    return out.reshape(R, S, D)


# speedup vs baseline: 1.3544x; 1.2612x over previous
import functools

import jax
import jax.numpy as jnp
from jax import lax
from jax.experimental import pallas as pl
from jax.experimental.pallas import tpu as pltpu
from jax.experimental.pallas import tpu_sc as plsc

SCALE = 8.0
NW = 32
LANES = 16
NBUF = 4
NOBUF = 2
C = 128


def kernel(input, table):
    R, S = input.shape
    B = R * S
    V, D = table.shape
    BW = B // NW
    NCHUNK = BW // C

    idx = input.reshape(B // C, C)
    tbl = table.reshape(V // 2, 2 * D)

    mesh = plsc.VectorSubcoreMesh(core_axis_name="c", subcore_axis_name="s")

    @functools.partial(
        pl.kernel,
        mesh=mesh,
        out_type=jax.ShapeDtypeStruct((B, D), jnp.float32),
        scratch_types=[
            pltpu.VMEM((NCHUNK, C), jnp.int32),
            [pltpu.VMEM((C,), jnp.int32) for _ in range(NBUF)],
            [pltpu.VMEM((C, 2 * D), jnp.float32) for _ in range(NBUF)],
            [pltpu.VMEM((C, D), jnp.float32) for _ in range(NOBUF)],
            [pltpu.SemaphoreType.DMA for _ in range(NBUF)],
            [pltpu.SemaphoreType.DMA for _ in range(NOBUF)],
        ],
        compiler_params=pltpu.CompilerParams(use_tc_tiling_on_sc=True),
    )
    def emb(idx_hbm, tbl_hbm, out_hbm, idx_v, sidx, gbufs, obufs,
            gsems, ssems):
        wid = lax.axis_index("s") * 2 + lax.axis_index("c")
        base = wid * BW
        pltpu.sync_copy(idx_hbm.at[pl.ds(wid * NCHUNK, NCHUNK)], idx_v)

        def fire(c, t):
            for g in range(C // LANES):
                sl = pl.ds(g * LANES, LANES)
                sidx[t][sl] = lax.shift_right_logical(idx_v[c, sl], 1)
            pltpu.async_copy(tbl_hbm.at[sidx[t]], gbufs[t], gsems[t])

        def gdrain(t):
            pltpu.make_async_copy(
                tbl_hbm.at[pl.ds(0, C)], gbufs[t], gsems[t]).wait()

        def sdrain(u):
            pltpu.make_async_copy(
                out_hbm.at[pl.ds(0, C)], obufs[u], ssems[u]).wait()

        def process(c, t, u):
            def group_body(g, carry):
                offv = (idx_v[c, pl.ds(g * LANES, LANES)] & 1) * D
                for k in range(LANES):
                    r = g * LANES + k
                    off = offv[k]
                    for s in range(D // LANES):
                        src = pl.ds(off + s * LANES, LANES)
                        dst = pl.ds(s * LANES, LANES)
                        obufs[u][r, dst] = gbufs[t][r, src] * SCALE
                return carry
            lax.fori_loop(0, C // LANES, group_body, 0)

        for t in range(NBUF - 1):
            fire(t, t)

        def body(i, carry):
            for t in range(NBUF):
                c = i * NBUF + t
                u = t % NOBUF
                gdrain(t)

                @pl.when(c >= NOBUF)
                def _():
                    sdrain(u)

                process(c, t, u)
                pltpu.async_copy(
                    obufs[u], out_hbm.at[pl.ds(base + c * C, C)], ssems[u])
                nt = (t + NBUF - 1) % NBUF
                nc = c + NBUF - 1

                @pl.when(nc <= NCHUNK - 1)
                def _():
                    fire(nc, nt)
            return carry

        lax.fori_loop(0, NCHUNK // NBUF, body, 0)
        for u in range(NOBUF):
            sdrain(u)

    out = emb(idx, tbl)
    return out.reshape(R, S, D)


# speedup vs baseline: 1.3806x; 1.0193x over previous
import functools

import jax
import jax.numpy as jnp
from jax import lax
from jax.experimental import pallas as pl
from jax.experimental.pallas import tpu as pltpu
from jax.experimental.pallas import tpu_sc as plsc

SCALE = 8.0
NW = 32
LANES = 16
NBUF = 4
NOBUF = 2
C = 128


def kernel(input, table):
    R, S = input.shape
    B = R * S
    V, D = table.shape
    BW = B // NW
    NCHUNK = BW // C

    idx = input.reshape(B // C, C)

    mesh = plsc.VectorSubcoreMesh(core_axis_name="c", subcore_axis_name="s")

    @functools.partial(
        pl.kernel,
        mesh=mesh,
        out_type=jax.ShapeDtypeStruct((B, 2 * D), jnp.float32),
        scratch_types=[
            pltpu.VMEM((NCHUNK, C), jnp.int32),
            [pltpu.VMEM((C, D), jnp.float32) for _ in range(NBUF)],
            [pltpu.VMEM((C, 2 * D), jnp.float32) for _ in range(NOBUF)],
            [pltpu.SemaphoreType.DMA for _ in range(NBUF)],
            [pltpu.SemaphoreType.DMA for _ in range(NOBUF)],
        ],
        compiler_params=pltpu.CompilerParams(use_tc_tiling_on_sc=False),
    )
    def emb(idx_hbm, table_hbm, out_hbm, idx_v, gbufs, obufs, gsems, ssems):
        wid = lax.axis_index("s") * 2 + lax.axis_index("c")
        base = wid * BW
        pltpu.sync_copy(idx_hbm.at[pl.ds(wid * NCHUNK, NCHUNK)], idx_v)

        def fire(c, t):
            pltpu.async_copy(table_hbm.at[idx_v.at[c]], gbufs[t], gsems[t])

        def gdrain(t):
            pltpu.make_async_copy(
                table_hbm.at[pl.ds(0, C)], gbufs[t], gsems[t]).wait()

        def sdrain(u):
            pltpu.make_async_copy(
                out_hbm.at[pl.ds(0, C)], obufs[u], ssems[u]).wait()

        def process(t, u):
            def row_body(r, carry):
                for s in range(D // LANES):
                    sl = pl.ds(s * LANES, LANES)
                    obufs[u][r, sl] = gbufs[t][r, sl] * SCALE
                return carry
            lax.fori_loop(0, C, row_body, 0)

        for t in range(NBUF - 1):
            fire(t, t)

        def body(i, carry):
            for t in range(NBUF):
                c = i * NBUF + t
                u = t % NOBUF
                gdrain(t)

                @pl.when(c >= NOBUF)
                def _():
                    sdrain(u)

                process(t, u)
                pltpu.async_copy(
                    obufs[u], out_hbm.at[pl.ds(base + c * C, C)], ssems[u])
                nt = (t + NBUF - 1) % NBUF
                nc = c + NBUF - 1

                @pl.when(nc <= NCHUNK - 1)
                def _():
                    fire(nc, nt)
            return carry

        lax.fori_loop(0, NCHUNK // NBUF, body, 0)
        for u in range(NOBUF):
            sdrain(u)

    out = emb(idx, table)
    return out[:, :D].reshape(R, S, D)


# speedup vs baseline: 1.8806x; 1.3622x over previous
import functools

import jax
import jax.numpy as jnp
from jax import lax
from jax.experimental import pallas as pl
from jax.experimental.pallas import tpu as pltpu
from jax.experimental.pallas import tpu_sc as plsc

SCALE = 8.0
NW = 32
LANES = 16
NBUF = 4
NOBUF = 2
C = 128


def kernel(input, table):
    R, S = input.shape
    B = R * S
    V, D = table.shape
    BW = B // NW
    NCHUNK = BW // C

    idx = input.reshape(B // C, C)

    mesh = plsc.VectorSubcoreMesh(core_axis_name="c", subcore_axis_name="s")

    @functools.partial(
        pl.kernel,
        mesh=mesh,
        out_type=jax.ShapeDtypeStruct((B, 2 * D), jnp.float32),
        scratch_types=[
            pltpu.VMEM((NCHUNK, C), jnp.int32),
            [pltpu.VMEM((C, D), jnp.float32) for _ in range(NBUF)],
            [pltpu.SemaphoreType.DMA for _ in range(NBUF)],
            [pltpu.SemaphoreType.DMA for _ in range(NBUF)],
        ],
        compiler_params=pltpu.CompilerParams(use_tc_tiling_on_sc=False),
    )
    def emb(idx_hbm, table_hbm, out_hbm, idx_v, gbufs, gsems, ssems):
        wid = lax.axis_index("s") * 2 + lax.axis_index("c")
        base = wid * BW
        pltpu.sync_copy(idx_hbm.at[pl.ds(wid * NCHUNK, NCHUNK)], idx_v)

        def fire(c, t):
            pltpu.async_copy(table_hbm.at[idx_v.at[c]], gbufs[t], gsems[t])

        def gdrain(t):
            pltpu.make_async_copy(
                table_hbm.at[pl.ds(0, C)], gbufs[t], gsems[t]).wait()

        def sdrain(u):
            pltpu.make_async_copy(
                out_hbm.at[pl.ds(0, C), pl.ds(0, D)], gbufs[u], ssems[u]).wait()

        def process(t):
            def row_body(r, carry):
                for s in range(D // LANES):
                    sl = pl.ds(s * LANES, LANES)
                    gbufs[t][r, sl] = gbufs[t][r, sl] * SCALE
                return carry
            lax.fori_loop(0, C, row_body, 0)

        for t in range(NBUF - 1):
            fire(t, t)

        def body(i, carry):
            for t in range(NBUF):
                c = i * NBUF + t
                gdrain(t)
                process(t)
                pltpu.async_copy(
                    gbufs[t],
                    out_hbm.at[pl.ds(base + c * C, C), pl.ds(0, D)],
                    ssems[t])
                nt = (t + NBUF - 1) % NBUF
                nc = c + NBUF - 1

                @pl.when(jnp.logical_and(c >= 1, nc <= NCHUNK - 1))
                def _():
                    sdrain(nt)

                @pl.when(nc <= NCHUNK - 1)
                def _():
                    fire(nc, nt)
            return carry

        lax.fori_loop(0, NCHUNK // NBUF, body, 0)
        for t in range(NBUF):
            sdrain(t)

    out = emb(idx, table)
    return out[:, :D].reshape(R, S, D)
